# baseline (speedup 1.0000x reference)
import jax
import jax.numpy as jnp
from jax import lax
from jax.experimental import pallas as pl
from jax.experimental.pallas import tpu as pltpu
from jax.experimental.pallas import tpu_sc as plsc

N = 100000
NNZ = 3200000
L = 5
THRESHOLD = 0.01
SLOPE = 5.0

NC, NS = 2, 16
NW = NC * NS
LANES = 16

NP = 100352
NBLK = 52
BLK = 2048
EPW = NBLK * BLK
NNZ_P = NW * EPW
SUB = NP // NS


def _spmv_body(x_hbm, cols_hbm, rows_hbm, vals_hbm, yp_hbm,
               x_sh, y_sh,
               cols_v0, cols_v1, vals_v0, vals_v1, xg_v0, xg_v1,
               rows_v0, rows_v1, rows_v2, rows_v3,
               w_v0, w_v1, w_v2, w_v3, zb_v,
               sem_in0, sem_in1, sem_g0, sem_g1,
               sem_s0, sem_s1, sem_s2, sem_s3):
    cid = lax.axis_index("c")
    sid = lax.axis_index("s")
    wid = cid * NS + sid

    cols_v = (cols_v0, cols_v1)
    vals_v = (vals_v0, vals_v1)
    xg_v = (xg_v0, xg_v1)
    rows_v = (rows_v0, rows_v1, rows_v2, rows_v3)
    w_v = (w_v0, w_v1, w_v2, w_v3)
    sem_in = (sem_in0, sem_in1)
    sem_g = (sem_g0, sem_g1)
    sem_s = (sem_s0, sem_s1, sem_s2, sem_s3)

    pltpu.sync_copy(x_hbm.at[pl.ds(sid * SUB, SUB)],
                    x_sh.at[pl.ds(sid * SUB, SUB)])

    @pl.loop(0, SUB, step=LANES)
    def _(i):
        zb_v[pl.ds(i, LANES)] = jnp.zeros((LANES,), jnp.float32)

    pltpu.sync_copy(zb_v, y_sh.at[pl.ds(sid * SUB, SUB)])
    plsc.subcore_barrier()

    e_base = wid * EPW

    def stage(p2, p4, bb):
        e0 = e_base + bb * BLK
        pltpu.async_copy(cols_hbm.at[pl.ds(e0, BLK)], cols_v[p2], sem_in[p2])
        pltpu.async_copy(vals_hbm.at[pl.ds(e0, BLK)], vals_v[p2], sem_in[p2])
        pltpu.async_copy(rows_hbm.at[pl.ds(e0, BLK)], rows_v[p4], sem_in[p2])

    def drain_stage(p2, p4, bb):
        e0 = e_base + bb * BLK
        pltpu.make_async_copy(cols_hbm.at[pl.ds(e0, BLK)], cols_v[p2], sem_in[p2]).wait()
        pltpu.make_async_copy(vals_hbm.at[pl.ds(e0, BLK)], vals_v[p2], sem_in[p2]).wait()
        pltpu.make_async_copy(rows_hbm.at[pl.ds(e0, BLK)], rows_v[p4], sem_in[p2]).wait()

    def fire_gather(p2):
        pltpu.make_async_copy(x_sh.at[cols_v[p2]], xg_v[p2], sem_g[p2]).start()

    def wait_gather(p2):
        pltpu.make_async_copy(x_sh.at[cols_v[p2]], xg_v[p2], sem_g[p2]).wait()

    def fire_scatter(p4):
        pltpu.make_async_copy(w_v[p4], y_sh.at[rows_v[p4]], sem_s[p4]).start(add=True)

    def drain_scatter(p4):
        pltpu.make_async_copy(w_v[p4], y_sh.at[rows_v[p4]], sem_s[p4]).wait()

    def multiply(p2, p4):
        @pl.loop(0, BLK, step=LANES)
        def _(i):
            w_v[p4][pl.ds(i, LANES)] = (
                vals_v[p2][pl.ds(i, LANES)] * xg_v[p2][pl.ds(i, LANES)])

    def phase(b, m4, drain_sc, do_next, do_stage2):
        p2, p2n = (m4 % 2), ((m4 + 1) % 2)
        p4, p4n1, p4n2, p4d = m4, ((m4 + 1) % 4), ((m4 + 2) % 4), ((m4 - 2) % 4)
        wait_gather(p2)
        if do_next:
            drain_stage(p2n, p4n1, b + 1)
            fire_gather(p2n)
        if drain_sc:
            drain_scatter(p4d)
        multiply(p2, p4)
        fire_scatter(p4)
        if do_stage2:
            stage(p2, p4n2, b + 2)

    stage(0, 0, 0)
    stage(1, 1, 1)
    drain_stage(0, 0, 0)
    fire_gather(0)
    phase(0, 0, drain_sc=False, do_next=True, do_stage2=True)
    phase(1, 1, drain_sc=False, do_next=True, do_stage2=True)

    @pl.loop(2, NBLK - 2, step=4)
    def _(bb):
        for o in range(4):
            phase(bb + o, (2 + o) % 4, drain_sc=True, do_next=True,
                  do_stage2=True)

    phase(NBLK - 2, (NBLK - 2) % 4, drain_sc=True, do_next=True,
          do_stage2=False)
    phase(NBLK - 1, (NBLK - 1) % 4, drain_sc=True, do_next=False,
          do_stage2=False)

    drain_scatter((NBLK - 2) % 4)
    drain_scatter((NBLK - 1) % 4)

    plsc.subcore_barrier()
    pltpu.sync_copy(y_sh.at[pl.ds(sid * SUB, SUB)],
                    yp_hbm.at[cid, pl.ds(sid * SUB, SUB)])


_spmv = pl.kernel(
    _spmv_body,
    out_type=jax.ShapeDtypeStruct((NC, NP), jnp.float32),
    mesh=plsc.VectorSubcoreMesh(core_axis_name="c", subcore_axis_name="s"),
    scratch_types=[
        pltpu.VMEM_SHARED((NP,), jnp.float32),
        pltpu.VMEM_SHARED((NP,), jnp.float32),
        pltpu.VMEM((BLK,), jnp.int32),
        pltpu.VMEM((BLK,), jnp.int32),
        pltpu.VMEM((BLK,), jnp.float32),
        pltpu.VMEM((BLK,), jnp.float32),
        pltpu.VMEM((BLK,), jnp.float32),
        pltpu.VMEM((BLK,), jnp.float32),
        pltpu.VMEM((BLK,), jnp.int32),
        pltpu.VMEM((BLK,), jnp.int32),
        pltpu.VMEM((BLK,), jnp.int32),
        pltpu.VMEM((BLK,), jnp.int32),
        pltpu.VMEM((BLK,), jnp.float32),
        pltpu.VMEM((BLK,), jnp.float32),
        pltpu.VMEM((BLK,), jnp.float32),
        pltpu.VMEM((BLK,), jnp.float32),
        pltpu.VMEM((SUB,), jnp.float32),
        pltpu.SemaphoreType.DMA,
        pltpu.SemaphoreType.DMA,
        pltpu.SemaphoreType.DMA,
        pltpu.SemaphoreType.DMA,
        pltpu.SemaphoreType.DMA,
        pltpu.SemaphoreType.DMA,
        pltpu.SemaphoreType.DMA,
        pltpu.SemaphoreType.DMA,
    ],
)


def _act_body(yp_ref, m_ref, d_ref, o_ref):
    y = yp_ref[0] + yp_ref[1]
    y = jnp.where(y >= THRESHOLD, y, 0.0)
    a = jnp.tanh(jnp.maximum(SLOPE * y, 0.0))
    o_ref[...] = jnp.where(m_ref[...] > 0.0, d_ref[...], a)


_act = pl.pallas_call(
    _act_body,
    out_shape=jax.ShapeDtypeStruct((NP // 128, 128), jnp.float32),
)


def kernel(inputs, values, rows, cols, sensory_idx):
    zeros = jnp.zeros((NP,), jnp.float32)
    mask = zeros.at[sensory_idx].set(1.0)
    drives = [zeros.at[sensory_idx].set(inputs[:, t]) for t in range(L)]

    pad_e = NNZ_P - NNZ
    cols_p = jnp.pad(cols, (0, pad_e))
    rows_p = jnp.pad(rows, (0, pad_e))
    vals_p = jnp.pad(values, (0, pad_e))
    m2 = mask.reshape(NP // 128, 128)

    x = drives[0]
    acts = [x]
    for t in range(1, L):
        yp = _spmv(x, cols_p, rows_p, vals_p)
        xn = _act(yp.reshape(NC, NP // 128, 128), m2,
                  drives[t].reshape(NP // 128, 128))
        x = xn.reshape(NP)
        acts.append(x)
    return jnp.stack([a[:N] for a in acts], axis=1)


# speedup vs baseline: 1.0853x; 1.0853x over previous
import dataclasses

import jax
import jax.numpy as jnp
from jax import lax
from jax.experimental import pallas as pl
from jax.experimental.pallas import tpu as pltpu
from jax.experimental.pallas import tpu_sc as plsc

N = 100000
NNZ = 3200000
L = 5
THRESHOLD = 0.01
SLOPE = 5.0

NC, NS = 2, 16
NW = NC * NS
LANES = 16

NP = 100352
NBLK = 52
BLK = 2048
EPW = NBLK * BLK
NNZ_P = NW * EPW
SUB = NP // NS


def _spmv_body(x_hbm, cols_hbm, rows_hbm, vals_hbm, yp_hbm,
               y_sh, x_v,
               cols_v0, cols_v1, vals_v0, vals_v1,
               rows_v0, rows_v1, rows_v2, rows_v3,
               w_v0, w_v1, w_v2, w_v3,
               sem_in0, sem_in1, sem_s0, sem_s1, sem_s2, sem_s3):
    cid = lax.axis_index("c")
    sid = lax.axis_index("s")
    wid = cid * NS + sid

    cols_v = (cols_v0, cols_v1)
    vals_v = (vals_v0, vals_v1)
    rows_v = (rows_v0, rows_v1, rows_v2, rows_v3)
    w_v = (w_v0, w_v1, w_v2, w_v3)
    sem_in = (sem_in0, sem_in1)
    sem_s = (sem_s0, sem_s1, sem_s2, sem_s3)

    pltpu.async_copy(x_hbm.at[pl.ds(0, N)], x_v, sem_in0)

    @pl.loop(0, BLK, step=LANES)
    def _(i):
        w_v0[pl.ds(i, LANES)] = jnp.zeros((LANES,), jnp.float32)

    for k in range(3):
        pltpu.sync_copy(w_v0, y_sh.at[pl.ds(sid * SUB + k * BLK, BLK)])
    pltpu.sync_copy(w_v0.at[pl.ds(0, SUB - 3 * BLK)],
                    y_sh.at[pl.ds(sid * SUB + 3 * BLK, SUB - 3 * BLK)])
    pltpu.make_async_copy(x_hbm.at[pl.ds(0, N)], x_v, sem_in0).wait()
    plsc.subcore_barrier()

    e_base = wid * EPW

    def stage(p2, p4, bb):
        e0 = e_base + bb * BLK
        pltpu.async_copy(cols_hbm.at[pl.ds(e0, BLK)], cols_v[p2], sem_in[p2])
        pltpu.async_copy(vals_hbm.at[pl.ds(e0, BLK)], vals_v[p2], sem_in[p2])
        pltpu.async_copy(rows_hbm.at[pl.ds(e0, BLK)], rows_v[p4], sem_in[p2])

    def drain_stage(p2, p4, bb):
        e0 = e_base + bb * BLK
        pltpu.make_async_copy(cols_hbm.at[pl.ds(e0, BLK)], cols_v[p2], sem_in[p2]).wait()
        pltpu.make_async_copy(vals_hbm.at[pl.ds(e0, BLK)], vals_v[p2], sem_in[p2]).wait()
        pltpu.make_async_copy(rows_hbm.at[pl.ds(e0, BLK)], rows_v[p4], sem_in[p2]).wait()

    def fire_scatter(p4):
        pltpu.make_async_copy(w_v[p4], y_sh.at[rows_v[p4]], sem_s[p4]).start(add=True)

    def drain_scatter(p4):
        pltpu.make_async_copy(w_v[p4], y_sh.at[rows_v[p4]], sem_s[p4]).wait()

    def phase(b, m4, drain_sc, do_stage):
        p2, p2n = (m4 % 2), ((m4 + 1) % 2)
        p4, p4n1, p4d = m4, ((m4 + 1) % 4), ((m4 - 2) % 4)
        drain_stage(p2, p4, b)
        if drain_sc:
            drain_scatter(p4d)
        if do_stage:
            stage(p2n, p4n1, b + 1)

        @pl.loop(0, BLK, step=LANES)
        def _(i):
            idx = cols_v[p2][pl.ds(i, LANES)]
            xg = plsc.load_gather(x_v, [idx])
            w_v[p4][pl.ds(i, LANES)] = vals_v[p2][pl.ds(i, LANES)] * xg

        fire_scatter(p4)

    stage(0, 0, 0)
    phase(0, 0, drain_sc=False, do_stage=True)
    phase(1, 1, drain_sc=False, do_stage=True)

    @pl.loop(2, NBLK - 2, step=4)
    def _(bb):
        for o in range(4):
            phase(bb + o, (2 + o) % 4, drain_sc=True, do_stage=True)

    phase(NBLK - 2, (NBLK - 2) % 4, drain_sc=True, do_stage=True)
    phase(NBLK - 1, (NBLK - 1) % 4, drain_sc=True, do_stage=False)

    drain_scatter((NBLK - 2) % 4)
    drain_scatter((NBLK - 1) % 4)

    plsc.subcore_barrier()
    pltpu.sync_copy(y_sh.at[pl.ds(sid * SUB, SUB)],
                    yp_hbm.at[cid, pl.ds(sid * SUB, SUB)])


_spmv = pl.kernel(
    _spmv_body,
    out_type=jax.ShapeDtypeStruct((NC, NP), jnp.float32),
    mesh=plsc.VectorSubcoreMesh(core_axis_name="c", subcore_axis_name="s"),
    compiler_params=dataclasses.replace(pltpu.CompilerParams(),
                                        needs_layout_passes=False),
    scratch_types=[
        pltpu.VMEM_SHARED((NP,), jnp.float32),
        pltpu.VMEM((N,), jnp.float32),
        pltpu.VMEM((BLK,), jnp.int32),
        pltpu.VMEM((BLK,), jnp.int32),
        pltpu.VMEM((BLK,), jnp.float32),
        pltpu.VMEM((BLK,), jnp.float32),
        pltpu.VMEM((BLK,), jnp.int32),
        pltpu.VMEM((BLK,), jnp.int32),
        pltpu.VMEM((BLK,), jnp.int32),
        pltpu.VMEM((BLK,), jnp.int32),
        pltpu.VMEM((BLK,), jnp.float32),
        pltpu.VMEM((BLK,), jnp.float32),
        pltpu.VMEM((BLK,), jnp.float32),
        pltpu.VMEM((BLK,), jnp.float32),
        pltpu.SemaphoreType.DMA,
        pltpu.SemaphoreType.DMA,
        pltpu.SemaphoreType.DMA,
        pltpu.SemaphoreType.DMA,
        pltpu.SemaphoreType.DMA,
        pltpu.SemaphoreType.DMA,
    ],
)


def _act_body(yp_ref, m_ref, d_ref, o_ref):
    y = yp_ref[0] + yp_ref[1]
    y = jnp.where(y >= THRESHOLD, y, 0.0)
    a = jnp.tanh(jnp.maximum(SLOPE * y, 0.0))
    o_ref[...] = jnp.where(m_ref[...] > 0.0, d_ref[...], a)


_act = pl.pallas_call(
    _act_body,
    out_shape=jax.ShapeDtypeStruct((NP // 128, 128), jnp.float32),
)


def kernel(inputs, values, rows, cols, sensory_idx):
    zeros = jnp.zeros((NP,), jnp.float32)
    mask = zeros.at[sensory_idx].set(1.0)
    drives = [zeros.at[sensory_idx].set(inputs[:, t]) for t in range(L)]

    pad_e = NNZ_P - NNZ
    cols_p = jnp.pad(cols, (0, pad_e))
    rows_p = jnp.pad(rows, (0, pad_e))
    vals_p = jnp.pad(values, (0, pad_e))
    m2 = mask.reshape(NP // 128, 128)

    x = drives[0]
    acts = [x]
    for t in range(1, L):
        yp = _spmv(x, cols_p, rows_p, vals_p)
        xn = _act(yp.reshape(NC, NP // 128, 128), m2,
                  drives[t].reshape(NP // 128, 128))
        x = xn.reshape(NP)
        acts.append(x)
    return jnp.stack([a[:N] for a in acts], axis=1)
